# baseline (speedup 1.0000x reference)
import functools

import jax
import jax.numpy as jnp
from jax import lax
from jax.experimental import pallas as pl
from jax.experimental.pallas import tpu as pltpu
from jax.experimental.pallas import tpu_sc as plsc

N = 1024
E = 16384
NC = 2
NS = 16
L = 16
EPS = 1e-6
CHUNK = E // (NC * NS)
NB = CHUNK // 128
ROWS = N // NS


def _sc_edge_pass(P):
    ACOL = P - 16
    NV = P // L

    mesh = plsc.VectorSubcoreMesh(core_axis_name="c", subcore_axis_name="s")

    @functools.partial(
        pl.kernel,
        out_type=(
            jax.ShapeDtypeStruct((NC, N, P), jnp.float32),
            jax.ShapeDtypeStruct((NC, L), jnp.float32),
        ),
        mesh=mesh,
        compiler_params=pltpu.CompilerParams(
            needs_layout_passes=False, use_tc_tiling_on_sc=False),
        scratch_types=[
            pltpu.VMEM((NB, 128), jnp.int32),
            pltpu.VMEM((NB, 128), jnp.int32),
            pltpu.VMEM((CHUNK, P), jnp.float32),
            pltpu.VMEM((CHUNK, P), jnp.float32),
            pltpu.VMEM((ROWS, P), jnp.float32),
            pltpu.VMEM((L,), jnp.float32),
            pltpu.VMEM((NS, L), jnp.float32),
            pltpu.VMEM_SHARED((N, P), jnp.float32),
            pltpu.VMEM_SHARED((NS, L), jnp.float32),
            pltpu.SemaphoreType.DMA,
        ],
    )
    def body(tl_hbm, tr_hbm, src_hbm, tgt_hbm, a_out, m_out,
             src_v, tgt_v, tls_v, trt_v, z_v, mv_v, mall_v,
             acc_sh, max_sh, sem):
        c = lax.axis_index("c")
        s = lax.axis_index("s")

        row0 = c * (E // NC // 128) + s * NB
        pltpu.sync_copy(src_hbm.at[pl.ds(row0, NB)], src_v)
        pltpu.sync_copy(tgt_hbm.at[pl.ds(row0, NB)], tgt_v)

        cps = []
        for j in range(NB):
            cps.append(pltpu.async_copy(
                tl_hbm.at[src_v.at[j]], tls_v.at[pl.ds(j * 128, 128)], sem))
            cps.append(pltpu.async_copy(
                tr_hbm.at[tgt_v.at[j]], trt_v.at[pl.ds(j * 128, 128)], sem))

        zvec = jnp.zeros((L,), jnp.float32)

        def zero_row(r, _):
            for j in range(NV):
                z_v[r, pl.ds(j * L, L)] = zvec
            return 0

        lax.fori_loop(0, ROWS, zero_row, 0, unroll=4)
        pltpu.sync_copy(z_v, acc_sh.at[pl.ds(s * ROWS, ROWS)])

        for cp in cps:
            cp.wait()

        lastsl = pl.ds(ACOL, L)
        lane0 = lax.iota(jnp.int32, L) == 0
        neg = jnp.full((L,), -3.0e38, jnp.float32)

        def mx(e, m):
            v = tls_v[e, lastsl] + trt_v[e, lastsl]
            tls_v[e, lastsl] = v
            return jnp.maximum(m, jnp.where(lane0, v, neg))

        mvec = lax.fori_loop(0, CHUNK, mx, neg, unroll=8)

        mv_v[...] = mvec
        pltpu.sync_copy(mv_v, max_sh.at[s])
        plsc.subcore_barrier()
        pltpu.sync_copy(max_sh, mall_v)
        mall = neg
        for i in range(NS):
            mall = jnp.maximum(mall, mall_v[i, :])
        base = jnp.max(mall)
        basev = jnp.full((L,), base)

        def edge(e, _):
            vl = tls_v[e, lastsl]
            wv = jnp.exp(jnp.full((L,), vl[0]) - basev)
            for j in range(NV - 1):
                sl = pl.ds(j * L, L)
                v = tls_v[e, sl] + trt_v[e, sl]
                tls_v[e, sl] = jnp.maximum(v, 0.0) * wv
            tls_v[e, lastsl] = wv
            return 0

        lax.fori_loop(0, CHUNK, edge, 0, unroll=4)

        for j in range(NB):
            pltpu.sync_copy(tls_v.at[pl.ds(j * 128, 128)],
                            acc_sh.at[tgt_v.at[j]], add=True)
        plsc.subcore_barrier()

        pltpu.sync_copy(acc_sh.at[pl.ds(s * ROWS, ROWS)],
                        a_out.at[c].at[pl.ds(s * ROWS, ROWS)])

        @pl.when(s == 0)
        def _():
            mv_v[...] = jnp.full((L,), base)
            pltpu.sync_copy(mv_v, m_out.at[c])

    return body


def _dotT(x, w):
    return lax.dot_general(x, w, (((1,), (1,)), ((), ())),
                           preferred_element_type=jnp.float32)


def _tc_first(P):
    def body(x_ref, wl_ref, wr_ref, bl_ref, br_ref, tl_ref, tr_ref):
        x = x_ref[...]
        tl_ref[...] = _dotT(x, wl_ref[...]) + bl_ref[...]
        tr_ref[...] = _dotT(x, wr_ref[...]) + br_ref[...]

    return pl.pallas_call(
        body,
        out_shape=(jax.ShapeDtypeStruct((N, P), jnp.float32),) * 2,
    )


def _combine(a_ref, m_ref, do_in):
    m0 = m_ref[0, 0]
    m1 = m_ref[1, 0]
    mm = jnp.maximum(m0, m1)
    s0 = jnp.exp(m0 - mm)
    s1 = jnp.exp(m1 - mm)
    S = a_ref[0] * s0 + a_ref[1] * s1
    den = S[:, do_in:do_in + 1] + EPS
    return S[:, :do_in] / den


def _tc_combine(Pin, do_in, Pout):
    def body(a_ref, m_ref, wl_ref, wr_ref, bl_ref, br_ref, tl_ref, tr_ref):
        o = _combine(a_ref, m_ref, do_in)
        tl_ref[...] = _dotT(o, wl_ref[...]) + bl_ref[...]
        tr_ref[...] = _dotT(o, wr_ref[...]) + br_ref[...]

    return pl.pallas_call(
        body,
        out_shape=(jax.ShapeDtypeStruct((N, Pout), jnp.float32),) * 2,
        in_specs=[
            pl.BlockSpec(memory_space=pltpu.VMEM),
            pl.BlockSpec(memory_space=pltpu.SMEM),
            pl.BlockSpec(memory_space=pltpu.VMEM),
            pl.BlockSpec(memory_space=pltpu.VMEM),
            pl.BlockSpec(memory_space=pltpu.VMEM),
            pl.BlockSpec(memory_space=pltpu.VMEM),
        ],
    )


def _tc_final(Pin, do_in, G, C1, C2):
    def body(a_ref, m_ref, mg_ref, w1_ref, b1_ref, w2_ref, b2_ref, out_ref):
        h = _combine(a_ref, m_ref, do_in)
        g = lax.dot_general(mg_ref[...], h, (((0,), (0,)), ((), ())),
                            preferred_element_type=jnp.float32)
        t = jnp.maximum(_dotT(g, w1_ref[...]) + b1_ref[...], 0.0)
        out_ref[...] = _dotT(t, w2_ref[...]) + b2_ref[...]

    return pl.pallas_call(
        body,
        out_shape=jax.ShapeDtypeStruct((G, C2), jnp.float32),
        in_specs=[
            pl.BlockSpec(memory_space=pltpu.VMEM),
            pl.BlockSpec(memory_space=pltpu.SMEM),
            pl.BlockSpec(memory_space=pltpu.VMEM),
            pl.BlockSpec(memory_space=pltpu.VMEM),
            pl.BlockSpec(memory_space=pltpu.VMEM),
            pl.BlockSpec(memory_space=pltpu.VMEM),
            pl.BlockSpec(memory_space=pltpu.VMEM),
        ],
    )


def _pad_weights(Wf, bf, Ww, bw):
    do, di2 = Wf.shape
    di = di2 // 2
    P = do + 16
    WL = jnp.zeros((P, di), jnp.float32).at[:do].set(Wf[:, :di]).at[do].set(Ww[0, :di])
    WR = jnp.zeros((P, di), jnp.float32).at[:do].set(Wf[:, di:]).at[do].set(Ww[0, di:])
    bL = jnp.zeros((1, P), jnp.float32).at[0, :do].set(bf)
    bR = jnp.zeros((1, P), jnp.float32).at[0, do].set(bw[0])
    return WL, WR, bL, bR, P


def kernel(x, adj, src, tgt, Msrc, Mtgt, Mgraph,
           Wf1, bf1, Ww1, bw1, Wf2, bf2, Ww2, bw2, Wf3, bf3, Ww3, bw3,
           Wm1, bm1, Wm2, bm2):
    src2 = src.reshape(E // 128, 128)
    tgt2 = tgt.reshape(E // 128, 128)

    WL1, WR1, bL1, bR1, P1 = _pad_weights(Wf1, bf1, Ww1, bw1)
    WL2, WR2, bL2, bR2, P2 = _pad_weights(Wf2, bf2, Ww2, bw2)
    WL3, WR3, bL3, bR3, P3 = _pad_weights(Wf3, bf3, Ww3, bw3)
    do1 = Wf1.shape[0]
    do2 = Wf2.shape[0]
    do3 = Wf3.shape[0]

    tl1, tr1 = _tc_first(P1)(x, WL1, WR1, bL1, bR1)
    a1, m1 = _sc_edge_pass(P1)(tl1, tr1, src2, tgt2)
    tl2, tr2 = _tc_combine(P1, do1, P2)(a1, m1, WL2, WR2, bL2, bR2)
    a2, m2 = _sc_edge_pass(P2)(tl2, tr2, src2, tgt2)
    tl3, tr3 = _tc_combine(P2, do2, P3)(a2, m2, WL3, WR3, bL3, bR3)
    a3, m3 = _sc_edge_pass(P3)(tl3, tr3, src2, tgt2)

    G = Mgraph.shape[1]
    C2 = Wm2.shape[0]
    bm1_2 = bm1.reshape(1, -1)
    bm2_2 = bm2.reshape(1, -1)
    out = _tc_final(P3, do3, G, Wm1.shape[0], C2)(
        a3, m3, Mgraph, Wm1, bm1_2, Wm2, bm2_2)
    return out


# speedup vs baseline: 1.0574x; 1.0574x over previous
import functools

import jax
import jax.numpy as jnp
from jax import lax
from jax.experimental import pallas as pl
from jax.experimental.pallas import tpu as pltpu
from jax.experimental.pallas import tpu_sc as plsc

N = 1024
E = 16384
NC = 2
NS = 16
L = 16
EPS = 1e-6
CHUNK = E // (NC * NS)
NB = CHUNK // 128
ROWS = N // NS


def _sc_edge_pass(P):
    ACOL = P - 16
    NV = P // L

    mesh = plsc.VectorSubcoreMesh(core_axis_name="c", subcore_axis_name="s")

    @functools.partial(
        pl.kernel,
        out_type=(
            jax.ShapeDtypeStruct((NC, N, P), jnp.float32),
            jax.ShapeDtypeStruct((NC, L), jnp.float32),
        ),
        mesh=mesh,
        compiler_params=pltpu.CompilerParams(
            needs_layout_passes=False, use_tc_tiling_on_sc=False),
        scratch_types=[
            pltpu.VMEM((NB, 128), jnp.int32),
            pltpu.VMEM((NB, 128), jnp.int32),
            pltpu.VMEM((CHUNK, P), jnp.float32),
            pltpu.VMEM((CHUNK, P), jnp.float32),
            pltpu.VMEM((ROWS, P), jnp.float32),
            pltpu.VMEM((L,), jnp.float32),
            pltpu.VMEM((NS, L), jnp.float32),
            pltpu.VMEM_SHARED((N, P), jnp.float32),
            pltpu.VMEM_SHARED((NS, L), jnp.float32),
            pltpu.VMEM_SHARED((N, P), jnp.float32),
            pltpu.VMEM_SHARED((N, P), jnp.float32),
            pltpu.SemaphoreType.DMA,
        ],
    )
    def body(tl_hbm, tr_hbm, src_hbm, tgt_hbm, a_out, m_out,
             src_v, tgt_v, tls_v, trt_v, z_v, mv_v, mall_v,
             acc_sh, max_sh, tl_sh, tr_sh, sem):
        c = lax.axis_index("c")
        s = lax.axis_index("s")

        row0 = c * (E // NC // 128) + s * NB
        pltpu.sync_copy(src_hbm.at[pl.ds(row0, NB)], src_v)
        pltpu.sync_copy(tgt_hbm.at[pl.ds(row0, NB)], tgt_v)
        nsl = pl.ds(s * ROWS, ROWS)
        stg = [
            pltpu.async_copy(tl_hbm.at[nsl], tl_sh.at[nsl], sem),
            pltpu.async_copy(tr_hbm.at[nsl], tr_sh.at[nsl], sem),
        ]

        zvec = jnp.zeros((L,), jnp.float32)

        def zero_row(r, _):
            for j in range(NV):
                z_v[r, pl.ds(j * L, L)] = zvec
            return 0

        lax.fori_loop(0, ROWS, zero_row, 0, unroll=4)
        pltpu.sync_copy(z_v, acc_sh.at[nsl])
        for cp in stg:
            cp.wait()
        plsc.subcore_barrier()

        cps = []
        for j in range(NB):
            cps.append(pltpu.async_copy(
                tl_sh.at[src_v.at[j]], tls_v.at[pl.ds(j * 128, 128)], sem))
            cps.append(pltpu.async_copy(
                tr_sh.at[tgt_v.at[j]], trt_v.at[pl.ds(j * 128, 128)], sem))
        for cp in cps:
            cp.wait()

        lastsl = pl.ds(ACOL, L)
        lane0 = lax.iota(jnp.int32, L) == 0
        neg = jnp.full((L,), -3.0e38, jnp.float32)

        def mx(e, m):
            v = tls_v[e, lastsl] + trt_v[e, lastsl]
            tls_v[e, lastsl] = v
            return jnp.maximum(m, jnp.where(lane0, v, neg))

        mvec = lax.fori_loop(0, CHUNK, mx, neg, unroll=8)

        mv_v[...] = mvec
        pltpu.sync_copy(mv_v, max_sh.at[s])
        plsc.subcore_barrier()
        pltpu.sync_copy(max_sh, mall_v)
        mall = neg
        for i in range(NS):
            mall = jnp.maximum(mall, mall_v[i, :])
        base = jnp.max(mall)
        basev = jnp.full((L,), base)

        def edge(e, _):
            vl = tls_v[e, lastsl]
            wv = jnp.exp(jnp.full((L,), vl[0]) - basev)
            for j in range(NV - 1):
                sl = pl.ds(j * L, L)
                v = tls_v[e, sl] + trt_v[e, sl]
                tls_v[e, sl] = jnp.maximum(v, 0.0) * wv
            tls_v[e, lastsl] = wv
            return 0

        lax.fori_loop(0, CHUNK, edge, 0, unroll=4)

        for j in range(NB):
            pltpu.sync_copy(tls_v.at[pl.ds(j * 128, 128)],
                            acc_sh.at[tgt_v.at[j]], add=True)
        plsc.subcore_barrier()

        pltpu.sync_copy(acc_sh.at[pl.ds(s * ROWS, ROWS)],
                        a_out.at[c].at[pl.ds(s * ROWS, ROWS)])

        @pl.when(s == 0)
        def _():
            mv_v[...] = jnp.full((L,), base)
            pltpu.sync_copy(mv_v, m_out.at[c])

    return body


def _dotT(x, w):
    return lax.dot_general(x, w, (((1,), (1,)), ((), ())),
                           preferred_element_type=jnp.float32)


def _tc_first(P):
    def body(x_ref, wl_ref, wr_ref, bl_ref, br_ref, tl_ref, tr_ref):
        x = x_ref[...]
        tl_ref[...] = _dotT(x, wl_ref[...]) + bl_ref[...]
        tr_ref[...] = _dotT(x, wr_ref[...]) + br_ref[...]

    return pl.pallas_call(
        body,
        out_shape=(jax.ShapeDtypeStruct((N, P), jnp.float32),) * 2,
    )


def _combine(a_ref, m_ref, do_in):
    m0 = m_ref[0, 0]
    m1 = m_ref[1, 0]
    mm = jnp.maximum(m0, m1)
    s0 = jnp.exp(m0 - mm)
    s1 = jnp.exp(m1 - mm)
    S = a_ref[0] * s0 + a_ref[1] * s1
    den = S[:, do_in:do_in + 1] + EPS
    return S[:, :do_in] / den


def _tc_combine(Pin, do_in, Pout):
    def body(a_ref, m_ref, wl_ref, wr_ref, bl_ref, br_ref, tl_ref, tr_ref):
        o = _combine(a_ref, m_ref, do_in)
        tl_ref[...] = _dotT(o, wl_ref[...]) + bl_ref[...]
        tr_ref[...] = _dotT(o, wr_ref[...]) + br_ref[...]

    return pl.pallas_call(
        body,
        out_shape=(jax.ShapeDtypeStruct((N, Pout), jnp.float32),) * 2,
        in_specs=[
            pl.BlockSpec(memory_space=pltpu.VMEM),
            pl.BlockSpec(memory_space=pltpu.SMEM),
            pl.BlockSpec(memory_space=pltpu.VMEM),
            pl.BlockSpec(memory_space=pltpu.VMEM),
            pl.BlockSpec(memory_space=pltpu.VMEM),
            pl.BlockSpec(memory_space=pltpu.VMEM),
        ],
    )


def _tc_final(Pin, do_in, G, C1, C2):
    def body(a_ref, m_ref, mg_ref, w1_ref, b1_ref, w2_ref, b2_ref, out_ref):
        h = _combine(a_ref, m_ref, do_in)
        g = lax.dot_general(mg_ref[...], h, (((0,), (0,)), ((), ())),
                            preferred_element_type=jnp.float32)
        t = jnp.maximum(_dotT(g, w1_ref[...]) + b1_ref[...], 0.0)
        out_ref[...] = _dotT(t, w2_ref[...]) + b2_ref[...]

    return pl.pallas_call(
        body,
        out_shape=jax.ShapeDtypeStruct((G, C2), jnp.float32),
        in_specs=[
            pl.BlockSpec(memory_space=pltpu.VMEM),
            pl.BlockSpec(memory_space=pltpu.SMEM),
            pl.BlockSpec(memory_space=pltpu.VMEM),
            pl.BlockSpec(memory_space=pltpu.VMEM),
            pl.BlockSpec(memory_space=pltpu.VMEM),
            pl.BlockSpec(memory_space=pltpu.VMEM),
            pl.BlockSpec(memory_space=pltpu.VMEM),
        ],
    )


def _pad_weights(Wf, bf, Ww, bw):
    do, di2 = Wf.shape
    di = di2 // 2
    P = do + 16
    WL = jnp.zeros((P, di), jnp.float32).at[:do].set(Wf[:, :di]).at[do].set(Ww[0, :di])
    WR = jnp.zeros((P, di), jnp.float32).at[:do].set(Wf[:, di:]).at[do].set(Ww[0, di:])
    bL = jnp.zeros((1, P), jnp.float32).at[0, :do].set(bf)
    bR = jnp.zeros((1, P), jnp.float32).at[0, do].set(bw[0])
    return WL, WR, bL, bR, P


def kernel(x, adj, src, tgt, Msrc, Mtgt, Mgraph,
           Wf1, bf1, Ww1, bw1, Wf2, bf2, Ww2, bw2, Wf3, bf3, Ww3, bw3,
           Wm1, bm1, Wm2, bm2):
    src2 = src.reshape(E // 128, 128)
    tgt2 = tgt.reshape(E // 128, 128)

    WL1, WR1, bL1, bR1, P1 = _pad_weights(Wf1, bf1, Ww1, bw1)
    WL2, WR2, bL2, bR2, P2 = _pad_weights(Wf2, bf2, Ww2, bw2)
    WL3, WR3, bL3, bR3, P3 = _pad_weights(Wf3, bf3, Ww3, bw3)
    do1 = Wf1.shape[0]
    do2 = Wf2.shape[0]
    do3 = Wf3.shape[0]

    tl1, tr1 = _tc_first(P1)(x, WL1, WR1, bL1, bR1)
    a1, m1 = _sc_edge_pass(P1)(tl1, tr1, src2, tgt2)
    tl2, tr2 = _tc_combine(P1, do1, P2)(a1, m1, WL2, WR2, bL2, bR2)
    a2, m2 = _sc_edge_pass(P2)(tl2, tr2, src2, tgt2)
    tl3, tr3 = _tc_combine(P2, do2, P3)(a2, m2, WL3, WR3, bL3, bR3)
    a3, m3 = _sc_edge_pass(P3)(tl3, tr3, src2, tgt2)

    G = Mgraph.shape[1]
    C2 = Wm2.shape[0]
    bm1_2 = bm1.reshape(1, -1)
    bm2_2 = bm2.reshape(1, -1)
    out = _tc_final(P3, do3, G, Wm1.shape[0], C2)(
        a3, m3, Mgraph, Wm1, bm1_2, Wm2, bm2_2)
    return out


# speedup vs baseline: 1.4192x; 1.3422x over previous
import functools

import jax
import jax.numpy as jnp
from jax import lax
from jax.experimental import pallas as pl
from jax.experimental.pallas import tpu as pltpu
from jax.experimental.pallas import tpu_sc as plsc

N = 1024
E = 16384
NC = 2
NS = 16
L = 16
EPS = 1e-6
CHUNK = E // (NC * NS)
NB = CHUNK // 128
ROWS = N // NS


def _sc_edge_pass(P):
    ACOL = P - 16
    NV = P // L

    mesh = plsc.VectorSubcoreMesh(core_axis_name="c", subcore_axis_name="s")

    @functools.partial(
        pl.kernel,
        out_type=(
            jax.ShapeDtypeStruct((NC, N, P), jnp.float32),
            jax.ShapeDtypeStruct((NC, L), jnp.float32),
        ),
        mesh=mesh,
        compiler_params=pltpu.CompilerParams(
            needs_layout_passes=False, use_tc_tiling_on_sc=False),
        scratch_types=[
            pltpu.VMEM((NB, 128), jnp.int32),
            pltpu.VMEM((NB, 128), jnp.int32),
            pltpu.VMEM((CHUNK, P), jnp.float32),
            pltpu.VMEM((CHUNK, P), jnp.float32),
            pltpu.VMEM((ROWS, P), jnp.float32),
            pltpu.VMEM((L,), jnp.float32),
            pltpu.VMEM((NS, L), jnp.float32),
            pltpu.VMEM_SHARED((N, P), jnp.float32),
            pltpu.VMEM_SHARED((NS, L), jnp.float32),
            pltpu.VMEM_SHARED((N, P), jnp.float32),
            pltpu.VMEM_SHARED((N, P), jnp.float32),
            pltpu.SemaphoreType.DMA,
        ],
    )
    def body(tl_hbm, tr_hbm, src_hbm, tgt_hbm, a_out, m_out,
             src_v, tgt_v, tls_v, trt_v, z_v, mv_v, mall_v,
             acc_sh, max_sh, tl_sh, tr_sh, sem):
        c = lax.axis_index("c")
        s = lax.axis_index("s")

        row0 = c * (E // NC // 128) + s * NB
        pltpu.sync_copy(src_hbm.at[pl.ds(row0, NB)], src_v)
        pltpu.sync_copy(tgt_hbm.at[pl.ds(row0, NB)], tgt_v)
        nsl = pl.ds(s * ROWS, ROWS)
        stg = [
            pltpu.async_copy(tl_hbm.at[nsl], tl_sh.at[nsl], sem),
            pltpu.async_copy(tr_hbm.at[nsl], tr_sh.at[nsl], sem),
        ]

        zvec = jnp.zeros((L,), jnp.float32)

        @plsc.parallel_loop(0, ROWS, unroll=4)
        def _(r):
            for j in range(NV):
                z_v[r, pl.ds(j * L, L)] = zvec
        pltpu.sync_copy(z_v, acc_sh.at[nsl])
        for cp in stg:
            cp.wait()
        plsc.subcore_barrier()

        cps = []
        for j in range(NB):
            cps.append(pltpu.async_copy(
                tl_sh.at[src_v.at[j]], tls_v.at[pl.ds(j * 128, 128)], sem))
            cps.append(pltpu.async_copy(
                tr_sh.at[tgt_v.at[j]], trt_v.at[pl.ds(j * 128, 128)], sem))
        for cp in cps:
            cp.wait()

        lastsl = pl.ds(ACOL, L)
        lane0 = lax.iota(jnp.int32, L) == 0
        neg = jnp.full((L,), -3.0e38, jnp.float32)

        @plsc.parallel_loop(0, CHUNK, unroll=4, carry=neg)
        def mvec(e, m):
            v = tls_v[e, lastsl] + trt_v[e, lastsl]
            tls_v[e, lastsl] = v
            return jnp.maximum(m, jnp.where(lane0, v, neg))

        mv_v[...] = mvec
        pltpu.sync_copy(mv_v, max_sh.at[s])
        plsc.subcore_barrier()
        pltpu.sync_copy(max_sh, mall_v)
        mall = neg
        for i in range(NS):
            mall = jnp.maximum(mall, mall_v[i, :])
        base = jnp.max(mall)
        basev = jnp.full((L,), base)

        @plsc.parallel_loop(0, CHUNK, unroll=4)
        def _(e):
            vl = tls_v[e, lastsl]
            wv = jnp.exp(jnp.full((L,), vl[0]) - basev)
            for j in range(NV - 1):
                sl = pl.ds(j * L, L)
                v = tls_v[e, sl] + trt_v[e, sl]
                tls_v[e, sl] = jnp.maximum(v, 0.0) * wv
            tls_v[e, lastsl] = wv

        for j in range(NB):
            pltpu.sync_copy(tls_v.at[pl.ds(j * 128, 128)],
                            acc_sh.at[tgt_v.at[j]], add=True)
        plsc.subcore_barrier()

        pltpu.sync_copy(acc_sh.at[pl.ds(s * ROWS, ROWS)],
                        a_out.at[c].at[pl.ds(s * ROWS, ROWS)])

        @pl.when(s == 0)
        def _():
            mv_v[...] = jnp.full((L,), base)
            pltpu.sync_copy(mv_v, m_out.at[c])

    return body


def _dotT(x, w):
    return lax.dot_general(x, w, (((1,), (1,)), ((), ())),
                           preferred_element_type=jnp.float32)


def _tc_first(P):
    def body(x_ref, wl_ref, wr_ref, bl_ref, br_ref, tl_ref, tr_ref):
        x = x_ref[...]
        tl_ref[...] = _dotT(x, wl_ref[...]) + bl_ref[...]
        tr_ref[...] = _dotT(x, wr_ref[...]) + br_ref[...]

    return pl.pallas_call(
        body,
        out_shape=(jax.ShapeDtypeStruct((N, P), jnp.float32),) * 2,
    )


def _combine(a_ref, m_ref, do_in):
    m0 = m_ref[0, 0]
    m1 = m_ref[1, 0]
    mm = jnp.maximum(m0, m1)
    s0 = jnp.exp(m0 - mm)
    s1 = jnp.exp(m1 - mm)
    S = a_ref[0] * s0 + a_ref[1] * s1
    den = S[:, do_in:do_in + 1] + EPS
    return S[:, :do_in] / den


def _tc_combine(Pin, do_in, Pout):
    def body(a_ref, m_ref, wl_ref, wr_ref, bl_ref, br_ref, tl_ref, tr_ref):
        o = _combine(a_ref, m_ref, do_in)
        tl_ref[...] = _dotT(o, wl_ref[...]) + bl_ref[...]
        tr_ref[...] = _dotT(o, wr_ref[...]) + br_ref[...]

    return pl.pallas_call(
        body,
        out_shape=(jax.ShapeDtypeStruct((N, Pout), jnp.float32),) * 2,
        in_specs=[
            pl.BlockSpec(memory_space=pltpu.VMEM),
            pl.BlockSpec(memory_space=pltpu.SMEM),
            pl.BlockSpec(memory_space=pltpu.VMEM),
            pl.BlockSpec(memory_space=pltpu.VMEM),
            pl.BlockSpec(memory_space=pltpu.VMEM),
            pl.BlockSpec(memory_space=pltpu.VMEM),
        ],
    )


def _tc_final(Pin, do_in, G, C1, C2):
    def body(a_ref, m_ref, mg_ref, w1_ref, b1_ref, w2_ref, b2_ref, out_ref):
        h = _combine(a_ref, m_ref, do_in)
        g = lax.dot_general(mg_ref[...], h, (((0,), (0,)), ((), ())),
                            preferred_element_type=jnp.float32)
        t = jnp.maximum(_dotT(g, w1_ref[...]) + b1_ref[...], 0.0)
        out_ref[...] = _dotT(t, w2_ref[...]) + b2_ref[...]

    return pl.pallas_call(
        body,
        out_shape=jax.ShapeDtypeStruct((G, C2), jnp.float32),
        in_specs=[
            pl.BlockSpec(memory_space=pltpu.VMEM),
            pl.BlockSpec(memory_space=pltpu.SMEM),
            pl.BlockSpec(memory_space=pltpu.VMEM),
            pl.BlockSpec(memory_space=pltpu.VMEM),
            pl.BlockSpec(memory_space=pltpu.VMEM),
            pl.BlockSpec(memory_space=pltpu.VMEM),
            pl.BlockSpec(memory_space=pltpu.VMEM),
        ],
    )


def _pad_weights(Wf, bf, Ww, bw):
    do, di2 = Wf.shape
    di = di2 // 2
    P = do + 16
    WL = jnp.zeros((P, di), jnp.float32).at[:do].set(Wf[:, :di]).at[do].set(Ww[0, :di])
    WR = jnp.zeros((P, di), jnp.float32).at[:do].set(Wf[:, di:]).at[do].set(Ww[0, di:])
    bL = jnp.zeros((1, P), jnp.float32).at[0, :do].set(bf)
    bR = jnp.zeros((1, P), jnp.float32).at[0, do].set(bw[0])
    return WL, WR, bL, bR, P


def kernel(x, adj, src, tgt, Msrc, Mtgt, Mgraph,
           Wf1, bf1, Ww1, bw1, Wf2, bf2, Ww2, bw2, Wf3, bf3, Ww3, bw3,
           Wm1, bm1, Wm2, bm2):
    src2 = src.reshape(E // 128, 128)
    tgt2 = tgt.reshape(E // 128, 128)

    WL1, WR1, bL1, bR1, P1 = _pad_weights(Wf1, bf1, Ww1, bw1)
    WL2, WR2, bL2, bR2, P2 = _pad_weights(Wf2, bf2, Ww2, bw2)
    WL3, WR3, bL3, bR3, P3 = _pad_weights(Wf3, bf3, Ww3, bw3)
    do1 = Wf1.shape[0]
    do2 = Wf2.shape[0]
    do3 = Wf3.shape[0]

    tl1, tr1 = _tc_first(P1)(x, WL1, WR1, bL1, bR1)
    a1, m1 = _sc_edge_pass(P1)(tl1, tr1, src2, tgt2)
    tl2, tr2 = _tc_combine(P1, do1, P2)(a1, m1, WL2, WR2, bL2, bR2)
    a2, m2 = _sc_edge_pass(P2)(tl2, tr2, src2, tgt2)
    tl3, tr3 = _tc_combine(P2, do2, P3)(a2, m2, WL3, WR3, bL3, bR3)
    a3, m3 = _sc_edge_pass(P3)(tl3, tr3, src2, tgt2)

    G = Mgraph.shape[1]
    C2 = Wm2.shape[0]
    bm1_2 = bm1.reshape(1, -1)
    bm2_2 = bm2.reshape(1, -1)
    out = _tc_final(P3, do3, G, Wm1.shape[0], C2)(
        a3, m3, Mgraph, Wm1, bm1_2, Wm2, bm2_2)
    return out


# speedup vs baseline: 1.5947x; 1.1237x over previous
import functools

import jax
import jax.numpy as jnp
from jax import lax
from jax.experimental import pallas as pl
from jax.experimental.pallas import tpu as pltpu
from jax.experimental.pallas import tpu_sc as plsc

N = 1024
E = 16384
NC = 2
NS = 16
L = 16
EPS = 1e-6
CHUNK = E // (NC * NS)
NB = CHUNK // 128
ROWS = N // NS


def _sc_edge_pass(P):
    ACOL = P - 16
    NV = P // L

    mesh = plsc.VectorSubcoreMesh(core_axis_name="c", subcore_axis_name="s")

    @functools.partial(
        pl.kernel,
        out_type=(
            jax.ShapeDtypeStruct((NC, N, P), jnp.float32),
            jax.ShapeDtypeStruct((NC, L), jnp.float32),
        ),
        mesh=mesh,
        compiler_params=pltpu.CompilerParams(
            needs_layout_passes=False, use_tc_tiling_on_sc=False),
        scratch_types=[
            pltpu.VMEM((NB, 128), jnp.int32),
            pltpu.VMEM((NB, 128), jnp.int32),
            pltpu.VMEM((CHUNK, P), jnp.float32),
            pltpu.VMEM((CHUNK, P), jnp.float32),
            pltpu.VMEM((ROWS, P), jnp.float32),
            pltpu.VMEM((L,), jnp.float32),
            pltpu.VMEM((NS, L), jnp.float32),
            pltpu.VMEM_SHARED((N, P), jnp.float32),
            pltpu.VMEM_SHARED((NS, L), jnp.float32),
            pltpu.VMEM_SHARED((N, P), jnp.float32),
            pltpu.VMEM_SHARED((N, P), jnp.float32),
            pltpu.SemaphoreType.DMA,
            pltpu.SemaphoreType.DMA,
        ],
    )
    def body(tl_hbm, tr_hbm, src_hbm, tgt_hbm, a_out, m_out,
             src_v, tgt_v, tls_v, trt_v, z_v, mv_v, mall_v,
             acc_sh, max_sh, tl_sh, tr_sh, sem, sem2):
        c = lax.axis_index("c")
        s = lax.axis_index("s")

        row0 = c * (E // NC // 128) + s * NB
        pltpu.sync_copy(src_hbm.at[pl.ds(row0, NB)], src_v)
        pltpu.sync_copy(tgt_hbm.at[pl.ds(row0, NB)], tgt_v)
        nsl = pl.ds(s * ROWS, ROWS)
        stg = [
            pltpu.async_copy(tl_hbm.at[nsl], tl_sh.at[nsl], sem),
            pltpu.async_copy(tr_hbm.at[nsl], tr_sh.at[nsl], sem),
        ]

        zvec = jnp.zeros((L,), jnp.float32)

        @plsc.parallel_loop(0, ROWS, unroll=4)
        def _(r):
            for j in range(NV):
                z_v[r, pl.ds(j * L, L)] = zvec

        pltpu.sync_copy(z_v, acc_sh.at[nsl])
        for cp in stg:
            cp.wait()
        plsc.subcore_barrier()

        gcps = []
        for j in range(NB):
            bsl = pl.ds(j * 128, 128)
            gcps.append((pltpu.async_copy(tl_sh.at[src_v.at[j]],
                                          tls_v.at[bsl], sem),
                         pltpu.async_copy(tr_sh.at[tgt_v.at[j]],
                                          trt_v.at[bsl], sem)))

        lastsl = pl.ds(ACOL, L)
        lane0 = lax.iota(jnp.int32, L) == 0
        neg = jnp.full((L,), -3.0e38, jnp.float32)

        mvec = neg
        for j in range(NB):
            for cp in gcps[j]:
                cp.wait()

            @plsc.parallel_loop(j * 128, (j + 1) * 128, unroll=4, carry=mvec)
            def mvec(e, m):
                v = tls_v[e, lastsl] + trt_v[e, lastsl]
                tls_v[e, lastsl] = v
                return jnp.maximum(m, jnp.where(lane0, v, neg))

        mv_v[...] = mvec
        pltpu.sync_copy(mv_v, max_sh.at[s])
        plsc.subcore_barrier()
        pltpu.sync_copy(max_sh, mall_v)
        mall = neg
        for i in range(NS):
            mall = jnp.maximum(mall, mall_v[i, :])
        base = jnp.max(mall)
        basev = jnp.full((L,), base)

        scps = []
        for j in range(NB):
            @plsc.parallel_loop(j * 128, (j + 1) * 128, unroll=4)
            def _(e):
                vl = tls_v[e, lastsl]
                wv = jnp.exp(jnp.full((L,), vl[0]) - basev)
                for k in range(NV - 1):
                    sl = pl.ds(k * L, L)
                    v = tls_v[e, sl] + trt_v[e, sl]
                    tls_v[e, sl] = jnp.maximum(v, 0.0) * wv
                tls_v[e, lastsl] = wv

            scps.append(pltpu.async_copy(
                tls_v.at[pl.ds(j * 128, 128)],
                acc_sh.at[tgt_v.at[j]], sem2, add=True))

        for cp in scps:
            cp.wait()
        plsc.subcore_barrier()

        pltpu.sync_copy(acc_sh.at[nsl], a_out.at[c].at[nsl])

        @pl.when(s == 0)
        def _():
            mv_v[...] = basev
            pltpu.sync_copy(mv_v, m_out.at[c])

    return body


def _dotT(x, w):
    return lax.dot_general(x, w, (((1,), (1,)), ((), ())),
                           preferred_element_type=jnp.float32)


def _tables(x, wf_ref, ww_ref, bf_ref, bw_ref, tl_ref, tr_ref, di, do):
    wf = wf_ref[...]
    ww = ww_ref[...]
    zpad = jnp.zeros((15, di), jnp.float32)
    wl = jnp.concatenate([wf[:, :di], ww[:, :di], zpad], axis=0)
    wr = jnp.concatenate([wf[:, di:], ww[:, di:], zpad], axis=0)
    bl = jnp.concatenate(
        [bf_ref[...], jnp.zeros((1, 16), jnp.float32)], axis=1)
    br = jnp.concatenate(
        [jnp.zeros((1, do), jnp.float32), bw_ref[...],
         jnp.zeros((1, 15), jnp.float32)], axis=1)
    tl_ref[...] = _dotT(x, wl) + bl
    tr_ref[...] = _dotT(x, wr) + br


def _tc_first(di, do):
    P = do + 16

    def body(x_ref, wf_ref, ww_ref, bf_ref, bw_ref, tl_ref, tr_ref):
        _tables(x_ref[...], wf_ref, ww_ref, bf_ref, bw_ref,
                tl_ref, tr_ref, di, do)

    return pl.pallas_call(
        body,
        out_shape=(jax.ShapeDtypeStruct((N, P), jnp.float32),) * 2,
    )


def _combine(a_ref, m_ref, do_in):
    m0 = m_ref[0, 0]
    m1 = m_ref[1, 0]
    mm = jnp.maximum(m0, m1)
    s0 = jnp.exp(m0 - mm)
    s1 = jnp.exp(m1 - mm)
    S = a_ref[0] * s0 + a_ref[1] * s1
    den = S[:, do_in:do_in + 1] + EPS
    return S[:, :do_in] / den


def _vm(n=1):
    return [pl.BlockSpec(memory_space=pltpu.VMEM)] * n


def _tc_combine(do_in, di, do):
    P = do + 16

    def body(a_ref, m_ref, wf_ref, ww_ref, bf_ref, bw_ref, tl_ref, tr_ref):
        o = _combine(a_ref, m_ref, do_in)
        _tables(o, wf_ref, ww_ref, bf_ref, bw_ref, tl_ref, tr_ref, di, do)

    return pl.pallas_call(
        body,
        out_shape=(jax.ShapeDtypeStruct((N, P), jnp.float32),) * 2,
        in_specs=_vm() + [pl.BlockSpec(memory_space=pltpu.SMEM)] + _vm(4),
    )


def _tc_final(do_in, G, C2):
    def body(a_ref, m_ref, mg_ref, w1_ref, b1_ref, w2_ref, b2_ref, out_ref):
        h = _combine(a_ref, m_ref, do_in)
        g = lax.dot_general(mg_ref[...], h, (((0,), (0,)), ((), ())),
                            preferred_element_type=jnp.float32)
        t = jnp.maximum(_dotT(g, w1_ref[...]) + b1_ref[...], 0.0)
        out_ref[...] = _dotT(t, w2_ref[...]) + b2_ref[...]

    return pl.pallas_call(
        body,
        out_shape=jax.ShapeDtypeStruct((G, C2), jnp.float32),
        in_specs=_vm() + [pl.BlockSpec(memory_space=pltpu.SMEM)] + _vm(5),
    )


def kernel(x, adj, src, tgt, Msrc, Mtgt, Mgraph,
           Wf1, bf1, Ww1, bw1, Wf2, bf2, Ww2, bw2, Wf3, bf3, Ww3, bw3,
           Wm1, bm1, Wm2, bm2):
    src2 = src.reshape(E // 128, 128)
    tgt2 = tgt.reshape(E // 128, 128)
    do1, di1 = Wf1.shape[0], Wf1.shape[1] // 2
    do2, di2 = Wf2.shape[0], Wf2.shape[1] // 2
    do3, di3 = Wf3.shape[0], Wf3.shape[1] // 2

    tl1, tr1 = _tc_first(di1, do1)(
        x, Wf1, Ww1, bf1.reshape(1, -1), bw1.reshape(1, 1))
    a1, m1 = _sc_edge_pass(do1 + 16)(tl1, tr1, src2, tgt2)
    tl2, tr2 = _tc_combine(do1, di2, do2)(
        a1, m1, Wf2, Ww2, bf2.reshape(1, -1), bw2.reshape(1, 1))
    a2, m2 = _sc_edge_pass(do2 + 16)(tl2, tr2, src2, tgt2)
    tl3, tr3 = _tc_combine(do2, di3, do3)(
        a2, m2, Wf3, Ww3, bf3.reshape(1, -1), bw3.reshape(1, 1))
    a3, m3 = _sc_edge_pass(do3 + 16)(tl3, tr3, src2, tgt2)

    out = _tc_final(do3, Mgraph.shape[1], Wm2.shape[0])(
        a3, m3, Mgraph, Wm1, bm1.reshape(1, -1), Wm2, bm2.reshape(1, -1))
    return out
